# baseline (speedup 1.0000x reference)
import functools
import math

import jax
import jax.numpy as jnp
from jax import lax
from jax.experimental import pallas as pl
from jax.experimental.pallas import tpu as pltpu
from jax.experimental.pallas import tpu_sc as plsc

D_MODEL = 64
LANES = 16
NUM_CORES = 2
NUM_SUBCORES = 16
NUM_WORKERS = NUM_CORES * NUM_SUBCORES
BLK = 128
NBUF = 4


def _emb_body(n_steps, scale, xt_hbm, table_hbm, out_hbm, idx_v, raw_v, scl_v,
              isem, gsem, osem):
  cid = lax.axis_index("c")
  sid = lax.axis_index("s")
  wid = sid * NUM_CORES + cid

  def idx_start(t, b):
    pltpu.async_copy(xt_hbm.at[t, pl.ds(wid * BLK, BLK)], idx_v.at[b],
                     isem.at[b])

  def idx_wait(b):
    pltpu.make_async_copy(xt_hbm.at[0, pl.ds(0, BLK)], idx_v.at[b],
                          isem.at[b]).wait()

  def gather_start(b):
    pltpu.async_copy(table_hbm.at[idx_v.at[b]], raw_v.at[b], gsem.at[b])

  def gather_wait(b):
    pltpu.make_async_copy(table_hbm.at[idx_v.at[0]], raw_v.at[b],
                          gsem.at[b]).wait()

  def out_start(t, b):
    pltpu.async_copy(scl_v.at[b], out_hbm.at[t, :, wid], osem.at[b])

  def out_wait(b):
    pltpu.make_async_copy(scl_v.at[b], out_hbm.at[0, :, 0], osem.at[b]).wait()

  rows = lax.iota(jnp.int32, LANES)

  for t in range(3):
    idx_start(jnp.int32(t), t)
  idx_wait(0)
  gather_start(0)

  def group(g, carry):
    t0 = g * NBUF
    for b in range(NBUF):
      t = t0 + b

      @pl.when(t + 3 < n_steps)
      def _():
        idx_start(t + 3, (b + 3) % NBUF)

      @pl.when(t + 1 < n_steps)
      def _():
        bn = (b + 1) % NBUF
        idx_wait(bn)
        gather_start(bn)

      gather_wait(b)

      @pl.when(t >= NBUF)
      def _():
        out_wait(b)

      @plsc.parallel_loop(0, D_MODEL, unroll=4)
      def _(j):
        cols = jnp.full((LANES,), 0, jnp.int32) + j
        jt = j // 8
        jr = j % 8
        for k in range(BLK // LANES):
          v = plsc.load_gather(raw_v.at[b], [rows + (k * LANES), cols])
          scl_v[b, jt, jr, pl.ds(k * LANES, LANES)] = v * scale

      out_start(t, b)

    return carry

  lax.fori_loop(0, n_steps // NBUF, group, 0)

  for b in range(NBUF):
    out_wait(b)


def kernel(x, table):
  bsz, seq = x.shape
  vocab, d = table.shape
  assert d == D_MODEL
  assert bsz == NUM_WORKERS * BLK
  assert seq % NBUF == 0
  n_steps = seq

  xt = x.T.astype(jnp.int32)
  scale = jnp.float32(math.sqrt(d))

  mesh = plsc.VectorSubcoreMesh(
      core_axis_name="c", subcore_axis_name="s",
      num_cores=NUM_CORES, num_subcores=NUM_SUBCORES)

  o5 = pl.kernel(
      functools.partial(_emb_body, n_steps, scale),
      out_type=jax.ShapeDtypeStruct(
          (seq, d // 8, bsz // BLK, 8, BLK), jnp.float32),
      mesh=mesh,
      compiler_params=pltpu.CompilerParams(
          use_tc_tiling_on_sc=False, needs_layout_passes=False),
      scratch_types=[
          pltpu.VMEM((NBUF, BLK), jnp.int32),
          pltpu.VMEM((NBUF, BLK, D_MODEL), jnp.float32),
          pltpu.VMEM((NBUF, d // 8, 8, BLK), jnp.float32),
          pltpu.SemaphoreType.DMA((NBUF,)),
          pltpu.SemaphoreType.DMA((NBUF,)),
          pltpu.SemaphoreType.DMA((NBUF,)),
      ],
  )(xt, table)

  emb = jnp.transpose(o5, (2, 4, 0, 1, 3)).reshape(bsz, seq, d)
  return emb


# speedup vs baseline: 1.2297x; 1.2297x over previous
import functools
import math

import jax
import jax.numpy as jnp
from jax import lax
from jax.experimental import pallas as pl
from jax.experimental.pallas import tpu as pltpu
from jax.experimental.pallas import tpu_sc as plsc

D_MODEL = 64
LANES = 16
NUM_CORES = 2
NUM_SUBCORES = 16
NUM_WORKERS = NUM_CORES * NUM_SUBCORES
CHUNK = 128
NBUF = 4


def _emb_body(toks_per_w, scale, x_hbm, table_hbm, out_hbm, idx_v, raw_v,
              scl_v, gsem, osem):
  cid = lax.axis_index("c")
  sid = lax.axis_index("s")
  wid = sid * NUM_CORES + cid
  tok0 = wid * toks_per_w

  pltpu.sync_copy(x_hbm.at[pl.ds(tok0, toks_per_w)], idx_v)

  def gather_start(c, b):
    pltpu.async_copy(table_hbm.at[idx_v.at[pl.ds(c * CHUNK, CHUNK)]],
                     raw_v.at[b], gsem.at[b])

  def gather_wait(b):
    pltpu.make_async_copy(table_hbm.at[idx_v.at[pl.ds(0, CHUNK)]],
                          raw_v.at[b], gsem.at[b]).wait()

  def out_start(c, b):
    pltpu.async_copy(scl_v.at[b],
                     out_hbm.at[pl.ds(tok0 + c * CHUNK, CHUNK)], osem.at[b])

  def out_wait(b):
    pltpu.make_async_copy(scl_v.at[b], out_hbm.at[pl.ds(0, CHUNK)],
                          osem.at[b]).wait()

  for b in range(NBUF):
    gather_start(jnp.int32(b), b)

  def group(g, carry):
    for b in range(NBUF):
      c = g * NBUF + b
      gather_wait(b)

      @pl.when(g > 0)
      def _():
        out_wait(b)

      @plsc.parallel_loop(0, CHUNK, unroll=8)
      def _(r):
        for j in range(D_MODEL // LANES):
          sl = pl.ds(j * LANES, LANES)
          scl_v[b, r, sl] = raw_v[b, r, sl] * scale

      out_start(c, b)

      @pl.when(c + NBUF < toks_per_w // CHUNK)
      def _():
        gather_start(c + NBUF, b)

    return carry

  lax.fori_loop(0, toks_per_w // (CHUNK * NBUF), group, 0)

  for b in range(NBUF):
    out_wait(b)


def kernel(x, table):
  bsz, seq = x.shape
  vocab, d = table.shape
  assert d == D_MODEL
  n_tok = bsz * seq
  assert n_tok % (NUM_WORKERS * CHUNK * NBUF) == 0
  toks_per_w = n_tok // NUM_WORKERS

  scale = jnp.float32(math.sqrt(d))

  mesh = plsc.VectorSubcoreMesh(
      core_axis_name="c", subcore_axis_name="s",
      num_cores=NUM_CORES, num_subcores=NUM_SUBCORES)

  o2 = pl.kernel(
      functools.partial(_emb_body, toks_per_w, scale),
      out_type=jax.ShapeDtypeStruct((n_tok, 2 * d), jnp.float32),
      mesh=mesh,
      compiler_params=pltpu.CompilerParams(use_tc_tiling_on_sc=False),
      scratch_types=[
          pltpu.VMEM((toks_per_w,), jnp.int32),
          pltpu.VMEM((NBUF, CHUNK, d), jnp.float32),
          pltpu.VMEM((NBUF, CHUNK, 2 * d), jnp.float32),
          pltpu.SemaphoreType.DMA((NBUF,)),
          pltpu.SemaphoreType.DMA((NBUF,)),
      ],
  )(x.reshape(-1).astype(jnp.int32), table)

  return o2[:, :d].reshape(bsz, seq, d)


# speedup vs baseline: 1.3449x; 1.0937x over previous
import functools
import math

import jax
import jax.numpy as jnp
from jax import lax
from jax.experimental import pallas as pl
from jax.experimental.pallas import tpu as pltpu
from jax.experimental.pallas import tpu_sc as plsc

D_MODEL = 64
LANES = 16
NUM_CORES = 2
NUM_SUBCORES = 16
NUM_WORKERS = NUM_CORES * NUM_SUBCORES
CHUNK = 128
NBUF = 5


def _emb_body(toks_per_w, scale, x_hbm, table_hbm, out_hbm, idx_v, raw_v,
              scl_v, gsem, osem):
  cid = lax.axis_index("c")
  sid = lax.axis_index("s")
  wid = sid * NUM_CORES + cid
  tok0 = wid * toks_per_w

  pltpu.sync_copy(x_hbm.at[pl.ds(tok0, toks_per_w)], idx_v)

  def gather_start(c, b):
    pltpu.async_copy(table_hbm.at[idx_v.at[pl.ds(c * CHUNK, CHUNK)]],
                     raw_v.at[b], gsem.at[b])

  def gather_wait(b):
    pltpu.make_async_copy(table_hbm.at[idx_v.at[pl.ds(0, CHUNK)]],
                          raw_v.at[b], gsem.at[b]).wait()

  def out_start(c, b):
    pltpu.async_copy(scl_v.at[b],
                     out_hbm.at[pl.ds(tok0 + c * CHUNK, CHUNK),
                                pl.ds(0, D_MODEL)], osem.at[b])

  def out_wait(b):
    pltpu.make_async_copy(scl_v.at[b],
                          out_hbm.at[pl.ds(0, CHUNK), pl.ds(0, D_MODEL)],
                          osem.at[b]).wait()

  for b in range(NBUF):
    gather_start(jnp.int32(b), b)

  def group(g, carry):
    for b in range(NBUF):
      c = g * NBUF + b
      gather_wait(b)

      @pl.when(g > 0)
      def _():
        out_wait(b)

      @plsc.parallel_loop(0, CHUNK, unroll=8)
      def _(r):
        for j in range(D_MODEL // LANES):
          sl = pl.ds(j * LANES, LANES)
          scl_v[b, r, sl] = raw_v[b, r, sl] * scale

      out_start(c, b)

      @pl.when(c + NBUF < toks_per_w // CHUNK)
      def _():
        gather_start(c + NBUF, b)

    return carry

  lax.fori_loop(0, toks_per_w // (CHUNK * NBUF), group, 0)

  for b in range(NBUF):
    out_wait(b)


def kernel(x, table):
  bsz, seq = x.shape
  vocab, d = table.shape
  assert d == D_MODEL
  n_tok = bsz * seq
  assert n_tok % (NUM_WORKERS * CHUNK * NBUF) == 0
  toks_per_w = n_tok // NUM_WORKERS

  scale = jnp.float32(math.sqrt(d))

  mesh = plsc.VectorSubcoreMesh(
      core_axis_name="c", subcore_axis_name="s",
      num_cores=NUM_CORES, num_subcores=NUM_SUBCORES)

  o2 = pl.kernel(
      functools.partial(_emb_body, toks_per_w, scale),
      out_type=jax.ShapeDtypeStruct((n_tok, 2 * d), jnp.float32),
      mesh=mesh,
      compiler_params=pltpu.CompilerParams(use_tc_tiling_on_sc=False),
      scratch_types=[
          pltpu.VMEM((toks_per_w,), jnp.int32),
          pltpu.VMEM((NBUF, CHUNK, d), jnp.float32),
          pltpu.VMEM((NBUF, CHUNK, d), jnp.float32),
          pltpu.SemaphoreType.DMA((NBUF,)),
          pltpu.SemaphoreType.DMA((NBUF,)),
      ],
  )(x.reshape(-1).astype(jnp.int32), table)

  return o2[:, :d].reshape(bsz, seq, d)
